# Initial kernel scaffold; baseline (speedup 1.0000x reference)
import jax
import jax.numpy as jnp
from jax import lax
from jax.experimental import pallas as pl
from jax.experimental.pallas import tpu as pltpu
from jax.experimental.pallas import tpu_sc as plsc

N_NODES = 10000
N_PAD = 10240
D = 128
N_EDGES = 320000
NS = 16
E_PER_TILE = N_EDGES // NS
CHUNK = 80
N_CHUNKS = E_PER_TILE // CHUNK
ROWS_PER_TILE = N_PAD // NS
OCHUNK = 128
VECS = CHUNK // 16


def _sc_body(zr_hbm, zi_hbm, src_hbm, dst_hbm, or_hbm, oi_hbm,
             src_v, dst_v, gbuf0, gbuf1, counts_v, cseg_v, rcp_v, obuf,
             acc_sh, cst_sh, gsem0, gsem1, ssem0, ssem1):
    c = lax.axis_index("c")
    s = lax.axis_index("s")

    pltpu.sync_copy(src_hbm.at[s], src_v)
    pltpu.sync_copy(dst_hbm.at[s], dst_v)

    rows = pl.ds(s * ROWS_PER_TILE, ROWS_PER_TILE)

    @pl.when(c == 0)
    def _():
        pltpu.sync_copy(zr_hbm.at[rows], acc_sh.at[rows])

    @pl.when(c == 1)
    def _():
        pltpu.sync_copy(zi_hbm.at[rows], acc_sh.at[rows])

    zeros16 = jnp.zeros((16,), jnp.float32)

    def zbody(i, _):
        counts_v[pl.ds(i * 16, 16)] = zeros16
        return 0

    lax.fori_loop(0, N_PAD // 16, zbody, 0)

    def issue_gather(j, buf, sem):
        idx = src_v.at[j]

        @pl.when(c == 0)
        def _():
            pltpu.async_copy(zr_hbm.at[idx], buf, sem)

        @pl.when(c == 1)
        def _():
            pltpu.async_copy(zi_hbm.at[idx], buf, sem)

    issue_gather(0, gbuf0, gsem0)
    issue_gather(1, gbuf1, gsem1)

    plsc.subcore_barrier()

    ones16 = jnp.ones((16,), jnp.float32)

    def step(j, buf, gsem, ssem):
        pltpu.make_async_copy(zr_hbm.at[src_v.at[j]], buf, gsem).wait()
        dst_idx = dst_v.at[j]
        pltpu.async_copy(buf, acc_sh.at[dst_idx], ssem, add=True)
        for k in range(VECS):
            idx16 = dst_v[j, pl.ds(k * 16, 16)]
            plsc.addupdate_scatter(counts_v, [idx16], ones16)
        pltpu.make_async_copy(buf, acc_sh.at[dst_idx], ssem).wait()

        @pl.when(j < N_CHUNKS - 2)
        def _():
            issue_gather(j + 2, buf, gsem)

    def lbody(i, _):
        step(2 * i, gbuf0, gsem0, ssem0)
        step(2 * i + 1, gbuf1, gsem1, ssem1)
        return 0

    lax.fori_loop(0, N_CHUNKS // 2, lbody, 0)

    pltpu.sync_copy(counts_v, cst_sh.at[s])
    plsc.subcore_barrier()
    cols = pl.ds(s * ROWS_PER_TILE, ROWS_PER_TILE)
    pltpu.sync_copy(cst_sh.at[:, cols], cseg_v)

    def rbody(g, _):
        lanes = pl.ds(g * 16, 16)
        tot = ones16
        for t in range(NS):
            tot = tot + cseg_v[t, lanes]
        rcp_v[lanes] = 1.0 / tot
        return 0

    lax.fori_loop(0, ROWS_PER_TILE // 16, rbody, 0)

    def obody(k, _):
        r0 = s * ROWS_PER_TILE + k * OCHUNK
        orows = pl.ds(r0, OCHUNK)
        pltpu.sync_copy(acc_sh.at[orows], obuf)

        def rowbody(r, _):
            scale = jnp.full((16,), rcp_v[k * OCHUNK + r])
            for q in range(D // 16):
                lanes = pl.ds(q * 16, 16)
                obuf[r, lanes] = obuf[r, lanes] * scale
            return 0

        lax.fori_loop(0, OCHUNK, rowbody, 0)

        @pl.when(c == 0)
        def _():
            pltpu.sync_copy(obuf, or_hbm.at[orows])

        @pl.when(c == 1)
        def _():
            pltpu.sync_copy(obuf, oi_hbm.at[orows])

        return 0

    lax.fori_loop(0, ROWS_PER_TILE // OCHUNK, obody, 0)


def _run_sc(zr, zi, src3, dst3):
    mesh = plsc.VectorSubcoreMesh(
        core_axis_name="c", subcore_axis_name="s", num_cores=2,
        num_subcores=NS)

    out_type = (
        jax.ShapeDtypeStruct((N_PAD, D), jnp.float32),
        jax.ShapeDtypeStruct((N_PAD, D), jnp.float32),
    )
    scratch = [
        pltpu.VMEM((N_CHUNKS, CHUNK), jnp.int32),
        pltpu.VMEM((N_CHUNKS, CHUNK), jnp.int32),
        pltpu.VMEM((CHUNK, D), jnp.float32),
        pltpu.VMEM((CHUNK, D), jnp.float32),
        pltpu.VMEM((N_PAD,), jnp.float32),
        pltpu.VMEM((NS, ROWS_PER_TILE), jnp.float32),
        pltpu.VMEM((ROWS_PER_TILE,), jnp.float32),
        pltpu.VMEM((OCHUNK, D), jnp.float32),
        pltpu.VMEM_SHARED((N_PAD, D), jnp.float32),
        pltpu.VMEM_SHARED((NS, N_PAD), jnp.float32),
        pltpu.SemaphoreType.DMA,
        pltpu.SemaphoreType.DMA,
        pltpu.SemaphoreType.DMA,
        pltpu.SemaphoreType.DMA,
    ]

    fn = pl.kernel(_sc_body, out_type=out_type, mesh=mesh,
                   scratch_types=scratch)
    return fn(zr, zi, src3, dst3)


@jax.jit
def kernel(Z, edge_index):
    zr = jnp.real(Z)
    zi = jnp.imag(Z)
    pad = ((0, N_PAD - N_NODES), (0, 0))
    zr = jnp.pad(zr, pad)
    zi = jnp.pad(zi, pad)
    src3 = edge_index[0].reshape(NS, N_CHUNKS, CHUNK)
    dst3 = edge_index[1].reshape(NS, N_CHUNKS, CHUNK)
    o_r, o_i = _run_sc(zr, zi, src3, dst3)
    return lax.complex(o_r[:N_NODES], o_i[:N_NODES])


# speedup vs baseline: 9.7296x; 9.7296x over previous
import jax
import jax.numpy as jnp
from jax import lax
from jax.experimental import pallas as pl
from jax.experimental.pallas import tpu as pltpu
from jax.experimental.pallas import tpu_sc as plsc

N_NODES = 10000
N_PAD = 10240
D = 128
DH = D // 2
N_EDGES = 320000
NS = 16
E_PER_TILE = N_EDGES // NS
CHUNK = 80
N_CHUNKS = E_PER_TILE // CHUNK
ROWS_PER_TILE = N_PAD // NS
OCHUNK = 128
VECS = CHUNK // 16


def _sc_body(zr0_hbm, zr1_hbm, zi0_hbm, zi1_hbm, src_hbm, dst_hbm,
             or0_hbm, or1_hbm, oi0_hbm, oi1_hbm,
             src_v, dst_v, gbuf0, gbuf1, counts_v, cseg_v, rcp_v, obuf,
             acc_sh, cst_sh, gsem0, gsem1, ssem0, ssem1):
    c = lax.axis_index("c")
    s = lax.axis_index("s")

    pltpu.sync_copy(src_hbm.at[s], src_v)
    pltpu.sync_copy(dst_hbm.at[s], dst_v)

    rows = pl.ds(s * ROWS_PER_TILE, ROWS_PER_TILE)
    zeros16 = jnp.zeros((16,), jnp.float32)
    ones16 = jnp.ones((16,), jnp.float32)

    def zbody(i, _):
        counts_v[pl.ds(i * 16, 16)] = zeros16
        return 0

    lax.fori_loop(0, N_PAD // 16, zbody, 0)

    for half in range(2):
        z_hbm = (zr0_hbm, zr1_hbm)[half]
        w_hbm = (zi0_hbm, zi1_hbm)[half]
        o0_hbm = (or0_hbm, or1_hbm)[half]
        o1_hbm = (oi0_hbm, oi1_hbm)[half]

        @pl.when(c == 0)
        def _():
            pltpu.sync_copy(z_hbm.at[rows], acc_sh.at[rows])

        @pl.when(c == 1)
        def _():
            pltpu.sync_copy(w_hbm.at[rows], acc_sh.at[rows])

        def issue_gather(j, buf, sem):
            idx = src_v.at[j]

            @pl.when(c == 0)
            def _():
                pltpu.async_copy(z_hbm.at[idx], buf, sem)

            @pl.when(c == 1)
            def _():
                pltpu.async_copy(w_hbm.at[idx], buf, sem)

        issue_gather(0, gbuf0, gsem0)
        issue_gather(1, gbuf1, gsem1)

        plsc.subcore_barrier()

        def step(j, buf, gsem, ssem):
            pltpu.make_async_copy(z_hbm.at[src_v.at[j]], buf, gsem).wait()
            dst_idx = dst_v.at[j]
            pltpu.async_copy(buf, acc_sh.at[dst_idx], ssem, add=True)
            if half == 0:
                for k in range(VECS):
                    idx16 = dst_v[j, pl.ds(k * 16, 16)]
                    plsc.addupdate_scatter(counts_v, [idx16], ones16)
            pltpu.make_async_copy(buf, acc_sh.at[dst_idx], ssem).wait()

            @pl.when(j < N_CHUNKS - 2)
            def _():
                issue_gather(j + 2, buf, gsem)

        def lbody(i, _):
            step(2 * i, gbuf0, gsem0, ssem0)
            step(2 * i + 1, gbuf1, gsem1, ssem1)
            return 0

        lax.fori_loop(0, N_CHUNKS // 2, lbody, 0)

        if half == 0:
            pltpu.sync_copy(counts_v, cst_sh.at[s])
            plsc.subcore_barrier()
            cols = pl.ds(s * ROWS_PER_TILE, ROWS_PER_TILE)
            pltpu.sync_copy(cst_sh.at[:, cols], cseg_v)

            def rbody(g, _):
                lanes = pl.ds(g * 16, 16)
                tot = ones16
                for t in range(NS):
                    tot = tot + cseg_v[t, lanes]
                rcp_v[lanes] = 1.0 / tot
                return 0

            lax.fori_loop(0, ROWS_PER_TILE // 16, rbody, 0)
        else:
            plsc.subcore_barrier()

        def obody(k, _):
            r0 = s * ROWS_PER_TILE + k * OCHUNK
            orows = pl.ds(r0, OCHUNK)
            pltpu.sync_copy(acc_sh.at[orows], obuf)

            def rowbody(g, _):
                scales = rcp_v[pl.ds(k * OCHUNK + g * 16, 16)]
                for t in range(16):
                    r = g * 16 + t
                    scale = jnp.full((16,), scales[t])
                    for q in range(DH // 16):
                        lanes = pl.ds(q * 16, 16)
                        obuf[r, lanes] = obuf[r, lanes] * scale
                return 0

            lax.fori_loop(0, OCHUNK // 16, rowbody, 0)

            @pl.when(c == 0)
            def _():
                pltpu.sync_copy(obuf, o0_hbm.at[orows])

            @pl.when(c == 1)
            def _():
                pltpu.sync_copy(obuf, o1_hbm.at[orows])

            return 0

        lax.fori_loop(0, ROWS_PER_TILE // OCHUNK, obody, 0)

        if half == 0:
            plsc.subcore_barrier()


def _run_sc(zr0, zr1, zi0, zi1, src3, dst3):
    mesh = plsc.VectorSubcoreMesh(
        core_axis_name="c", subcore_axis_name="s", num_cores=2,
        num_subcores=NS)

    half_t = jax.ShapeDtypeStruct((N_PAD, DH), jnp.float32)
    out_type = (half_t, half_t, half_t, half_t)
    scratch = [
        pltpu.VMEM((N_CHUNKS, CHUNK), jnp.int32),
        pltpu.VMEM((N_CHUNKS, CHUNK), jnp.int32),
        pltpu.VMEM((CHUNK, DH), jnp.float32),
        pltpu.VMEM((CHUNK, DH), jnp.float32),
        pltpu.VMEM((N_PAD,), jnp.float32),
        pltpu.VMEM((NS, ROWS_PER_TILE), jnp.float32),
        pltpu.VMEM((ROWS_PER_TILE,), jnp.float32),
        pltpu.VMEM((OCHUNK, DH), jnp.float32),
        pltpu.VMEM_SHARED((N_PAD, DH), jnp.float32),
        pltpu.VMEM_SHARED((NS, N_PAD), jnp.float32),
        pltpu.SemaphoreType.DMA,
        pltpu.SemaphoreType.DMA,
        pltpu.SemaphoreType.DMA,
        pltpu.SemaphoreType.DMA,
    ]

    fn = pl.kernel(_sc_body, out_type=out_type, mesh=mesh,
                   scratch_types=scratch,
                   compiler_params=pltpu.CompilerParams(
                       needs_layout_passes=False,
                       use_tc_tiling_on_sc=False))
    return fn(zr0, zr1, zi0, zi1, src3, dst3)


@jax.jit
def kernel(Z, edge_index):
    zr = jnp.real(Z)
    zi = jnp.imag(Z)
    pad = ((0, N_PAD - N_NODES), (0, 0))
    zr = jnp.pad(zr, pad)
    zi = jnp.pad(zi, pad)
    src3 = edge_index[0].reshape(NS, N_CHUNKS, CHUNK)
    dst3 = edge_index[1].reshape(NS, N_CHUNKS, CHUNK)
    o_r0, o_r1, o_i0, o_i1 = _run_sc(
        zr[:, :DH], zr[:, DH:], zi[:, :DH], zi[:, DH:], src3, dst3)
    o_r = jnp.concatenate([o_r0[:N_NODES], o_r1[:N_NODES]], axis=1)
    o_i = jnp.concatenate([o_i0[:N_NODES], o_i1[:N_NODES]], axis=1)
    return lax.complex(o_r, o_i)
